# Initial kernel scaffold; baseline (speedup 1.0000x reference)
import jax
import jax.numpy as jnp
from jax.experimental import pallas as pl


def kernel(tk_codes, feat_grid):
    raise NotImplementedError("write your pallas kernel here")


# speedup vs baseline: 5.1950x; 5.1950x over previous
import functools

import jax
import jax.numpy as jnp
from jax import lax
from jax.experimental import pallas as pl
from jax.experimental.pallas import tpu as pltpu
from jax.experimental.pallas import tpu_sc as plsc

_B = 8
_C = 128
_H = 256
_W = 256
_T = 16384
_HW = _H * _W
_TOT = _B * _T

_TBLK = 1024


def _tc_transpose(feat):
    def body(in_ref, out_ref):
        out_ref[0] = in_ref[0].T

    return pl.pallas_call(
        body,
        grid=(_B, _HW // _TBLK),
        in_specs=[pl.BlockSpec((1, _C, _TBLK), lambda b, j: (b, 0, j))],
        out_specs=pl.BlockSpec((1, _TBLK, _C), lambda b, j: (b, j, 0)),
        out_shape=jax.ShapeDtypeStruct((_B, _HW, _C), jnp.float32),
    )(feat)


def _sc_gather(table, tk_flat):
    info = plsc.get_sparse_core_info()
    nw = info.num_cores * info.num_subcores
    per_w = _TOT // nw
    chunk = 512

    mesh = plsc.VectorSubcoreMesh(core_axis_name="c", subcore_axis_name="s")

    @functools.partial(
        pl.kernel,
        mesh=mesh,
        out_type=jax.ShapeDtypeStruct((_TOT, _C), jnp.float32),
        scratch_types=[
            pltpu.VMEM((2 * per_w,), jnp.int32),
            pltpu.VMEM((per_w,), jnp.int32),
            pltpu.VMEM((chunk, _C), jnp.float32),
            pltpu.SemaphoreType.DMA,
        ],
        compiler_params=pltpu.CompilerParams(needs_layout_passes=False),
    )
    def k(table_hbm, tk_hbm, out_hbm, tk_v, idx_v, rows_v, sem):
        wid = lax.axis_index("s") * info.num_cores + lax.axis_index("c")
        g0 = wid * per_w
        batch = g0 // _T
        row_base = batch * _HW

        pltpu.sync_copy(tk_hbm.at[pl.ds(2 * g0, 2 * per_w)], tk_v)

        lanes = lax.iota(jnp.int32, 16)

        def compute_idx(i, _):
            base = i * 32
            xv = plsc.load_gather(tk_v, [base + lanes * 2])
            yv = plsc.load_gather(tk_v, [base + lanes * 2 + 1])
            idx_v[pl.ds(i * 16, 16)] = row_base + yv * _W + xv
            return 0

        lax.fori_loop(0, per_w // 16, compute_idx, 0)

        def gather_chunk(j, _):
            t0 = j * chunk
            pltpu.async_copy(
                table_hbm.at[idx_v.at[pl.ds(t0, chunk)]], rows_v, sem
            ).wait()
            pltpu.sync_copy(rows_v, out_hbm.at[pl.ds(g0 + t0, chunk)])
            return 0

        lax.fori_loop(0, per_w // chunk, gather_chunk, 0)

    return k(table, tk_flat)


def kernel(tk_codes, feat_grid):
    tk_flat = tk_codes.astype(jnp.int32).reshape(-1)
    table = _tc_transpose(feat_grid.reshape(_B, _C, _HW)).reshape(_B * _HW, _C)
    out = _sc_gather(table, tk_flat)
    return out.reshape(_B, _T, _C)
